# Initial kernel scaffold; baseline (speedup 1.0000x reference)
import jax
import jax.numpy as jnp
from jax.experimental import pallas as pl


def kernel(x, edge_index, batch, node_attention, gin1_w1, gin1_b1, gin1_w2, gin1_b2, pool1_w, pool1_b, gin2_w1, gin2_b1, gin2_w2, gin2_b2, pool2_w, pool2_b, gin3_w1, gin3_b1, gin3_w2, gin3_b2, lin_w, lin_b):
    raise NotImplementedError("write your pallas kernel here")


# speedup vs baseline: 10.3269x; 10.3269x over previous
import functools

import jax
import jax.numpy as jnp
from jax import lax
from jax.experimental import pallas as pl
from jax.experimental.pallas import tpu as pltpu
from jax.experimental.pallas import tpu_sc as plsc

N = 10000
NPAD = 10240
E = 320000
G = 64
DF = 128
HID = 64
NC = 2
NS = 16
NW = NC * NS
EW = E // NW
MIN_SCORE = 1e-3
TOL = 1e-7
NEG = float("-inf")
F32 = jnp.float32


HALF = NPAD // 2
JUNK = HALF
ACC_R = 5376
EW2 = E // NS
SC_D = 128
SC_C = 400
ZC = 112


@functools.lru_cache(maxsize=None)
def _make_sc_scatter():
    RW = ACC_R // NS
    mesh = plsc.VectorSubcoreMesh(core_axis_name="c", subcore_axis_name="s")

    @functools.partial(
        pl.kernel,
        out_type=jax.ShapeDtypeStruct((NC, ACC_R, SC_D), F32),
        mesh=mesh,
        scratch_types=[
            pltpu.VMEM((SC_C,), jnp.int32),
            pltpu.VMEM((SC_C,), jnp.int32),
            pltpu.VMEM((SC_C, SC_D), F32),
            pltpu.VMEM((ZC, SC_D), F32),
            pltpu.VMEM_SHARED((ACC_R, SC_D), F32),
            pltpu.SemaphoreType.DMA,
        ],
    )
    def scatter_kernel(table_hbm, src_hbm, dsta_hbm, dstb_hbm, zeros_hbm,
                       out_hbm, src_v, dst_v, rows_v, zbuf_v, acc_s, sem):
        c = lax.axis_index("c")
        s = lax.axis_index("s")

        pltpu.sync_copy(zeros_hbm, zbuf_v)

        @pl.loop(0, RW // ZC)
        def _zero(i):
            pltpu.sync_copy(zbuf_v, acc_s.at[pl.ds(s * RW + i * ZC, ZC)])

        plsc.subcore_barrier()

        base = s * EW2

        def run(dst_hbm):
            @pl.loop(0, EW2 // SC_C)
            def _scat(j):
                off = base + j * SC_C
                pltpu.sync_copy(src_hbm.at[pl.ds(off, SC_C)], src_v)
                pltpu.sync_copy(dst_hbm.at[pl.ds(off, SC_C)], dst_v)
                pltpu.async_copy(table_hbm.at[src_v], rows_v, sem).wait()
                pltpu.sync_copy(rows_v, acc_s.at[dst_v], add=True)

        @pl.when(c == 0)
        def _ca():
            run(dsta_hbm)

        @pl.when(c == 1)
        def _cb():
            run(dstb_hbm)

        plsc.subcore_barrier()

        @pl.loop(0, RW // ZC)
        def _out(i):
            off = s * RW + i * ZC
            pltpu.sync_copy(acc_s.at[pl.ds(off, ZC)], zbuf_v)
            pltpu.sync_copy(zbuf_v, out_hbm.at[c, pl.ds(off, ZC)])

    return scatter_kernel


def _sc_scatter(table, src, dst2):
    halves = _make_sc_scatter()(table, src, dst2[0], dst2[1],
                                jnp.zeros((ZC, SC_D), F32))
    return jnp.concatenate([halves[0, :HALF], halves[1, :HALF]], axis=0)


RB = 1024


def _t1_body(x_ref, aa_ref, w1_ref, b1_ref, w2_ref, b2_ref, pw_ref,
             h_ref, xw_ref):
    h0 = x_ref[...] + aa_ref[...]
    h = jnp.maximum(jnp.dot(h0, w1_ref[...], preferred_element_type=F32)
                    + b1_ref[...], 0.0)
    h = jnp.dot(h, w2_ref[...], preferred_element_type=F32) + b2_ref[...]
    h = jnp.maximum(h, 0.0)
    h_ref[...] = h
    xw_ref[...] = jnp.dot(h, pw_ref[...], preferred_element_type=F32)


@functools.lru_cache(maxsize=None)
def _make_t1(din):
    grid = NPAD // RB
    row = lambda i: (i, 0)
    full = lambda i: (0, 0)
    return pl.pallas_call(
        _t1_body,
        grid=(grid,),
        in_specs=[
            pl.BlockSpec((RB, din), row),
            pl.BlockSpec((RB, din), row),
            pl.BlockSpec((din, HID), full),
            pl.BlockSpec((1, HID), full),
            pl.BlockSpec((HID, HID), full),
            pl.BlockSpec((1, HID), full),
            pl.BlockSpec((HID, 1), full),
        ],
        out_specs=[
            pl.BlockSpec((RB, HID), row),
            pl.BlockSpec((RB, 1), row),
        ],
        out_shape=[
            jax.ShapeDtypeStruct((NPAD, HID), F32),
            jax.ShapeDtypeStruct((NPAD, 1), F32),
        ],
    )


def _t1(x, agg, w1, b1, w2, b2, pw):
    return _make_t1(x.shape[1])(x, agg, w1, b1.reshape(1, HID),
                                w2, b2.reshape(1, HID), pw)


def _t2a_body(xw_ref, da_ref, u_ref, v_ref, dinv_ref):
    u = u_ref[...]
    deg = u * da_ref[...] + u
    dinv = jnp.where(deg > 0, 1.0 / jnp.sqrt(deg), 0.0)
    dinv_ref[...] = dinv
    v_ref[...] = dinv * u * xw_ref[...]


_t2a_call = None


def _t2a(xw, da, u):
    global _t2a_call
    if _t2a_call is None:
        _t2a_call = pl.pallas_call(
            _t2a_body,
            out_shape=[jax.ShapeDtypeStruct((NPAD, 1), F32),
                       jax.ShapeDtypeStruct((NPAD, 1), F32)],
        )
    return _t2a_call(xw, da, u)


CB2 = 1024
NCH2 = NPAD // CB2


def _t2b_body(sa_ref, dinv_ref, u_ref, xw_ref, pb_ref, batch_ref,
              h_ref, xk_ref, un_ref, sk_ref):
    iot = lax.broadcasted_iota(jnp.int32, (CB2, G + 1), 1)

    def chunk(i):
        sl = pl.ds(i * CB2, CB2)
        u = u_ref[sl, :]
        b = batch_ref[sl, :]
        dinv = dinv_ref[sl, :]
        oh = jnp.where(u > 0, b, G) == iot
        attn = (dinv * u * sa_ref[sl, :]
                + dinv * dinv * u * xw_ref[sl, :] + pb_ref[0, 0])
        return sl, u, b, oh, attn

    def p_max(i, m):
        _, _, _, oh, attn = chunk(i)
        return jnp.maximum(m, jnp.max(jnp.where(oh, attn, NEG), axis=0,
                                      keepdims=True))

    m = lax.fori_loop(0, NCH2, p_max, jnp.full((1, G + 1), NEG, F32))

    def score_chunk(i, d):
        sl, u, b, oh, attn = chunk(i)
        m_pn = jnp.sum(jnp.where(oh, m, 0.0), axis=1, keepdims=True)
        e = jnp.exp(attn - m_pn)
        if d is None:
            return sl, u, b, oh, e
        d_pn = jnp.sum(jnp.where(oh, d, 0.0), axis=1, keepdims=True)
        return sl, u, b, oh, e / d_pn

    def p_den(i, d):
        _, _, _, oh, e = score_chunk(i, None)
        return d + jnp.sum(jnp.where(oh, e, 0.0), axis=0, keepdims=True)

    d = lax.fori_loop(0, NCH2, p_den, jnp.zeros((1, G + 1), F32))

    def p_smax(i, smax):
        _, _, _, oh, score = score_chunk(i, d)
        return jnp.maximum(smax, jnp.max(jnp.where(oh, score, NEG), axis=0,
                                         keepdims=True))

    smax_seg = lax.fori_loop(0, NCH2, p_smax, jnp.full((1, G + 1), NEG, F32))

    def p_out(i, carry):
        sl, u, b, oh, score = score_chunk(i, d)
        oh_b = b == iot
        smax_pn = jnp.sum(jnp.where(oh_b, smax_seg, 0.0), axis=1,
                          keepdims=True) - TOL
        smin = jnp.minimum(smax_pn, MIN_SCORE)
        perm = (u > 0) & (score > smin)
        xk_ref[sl, :] = jnp.where(perm, h_ref[sl, :] * score, 0.0)
        un_ref[sl, :] = perm.astype(F32)
        sk_ref[sl, :] = jnp.where(perm, score, 0.0)
        return carry

    lax.fori_loop(0, NCH2, p_out, 0)


_t2b_call = None


def _t2b(sa, dinv, u, xw, pb, batchp, h):
    global _t2b_call
    if _t2b_call is None:
        _t2b_call = pl.pallas_call(
            _t2b_body,
            out_shape=[jax.ShapeDtypeStruct((NPAD, HID), F32),
                       jax.ShapeDtypeStruct((NPAD, 1), F32),
                       jax.ShapeDtypeStruct((NPAD, 1), F32)],
        )
    return _t2b_call(sa, dinv, u, xw, pb.reshape(1, 1), batchp, h)


RB3 = 256
NB3 = NPAD // RB3


def _t3_body(x_ref, aa_ref, w1_ref, b1_ref, w2_ref, b2_ref,
             lw_ref, lb_ref, batch_ref, m2_ref, s2_ref, t_ref,
             pred_ref, loss_ref, ratio_ref,
             gm_sc, kl_sc, cnt_sc, rat_sc):
    i = pl.program_id(0)

    @pl.when(i == 0)
    def _init():
        gm_sc[...] = jnp.full((G + 1, HID), NEG, F32)
        kl_sc[...] = jnp.zeros((1, G + 1), F32)
        cnt_sc[...] = jnp.zeros((1, G + 1), F32)
        rat_sc[...] = jnp.zeros((1, 1), F32)

    h0 = x_ref[...] + aa_ref[...]
    h = jnp.maximum(jnp.dot(h0, w1_ref[...], preferred_element_type=F32)
                    + b1_ref[...], 0.0)
    h = jnp.dot(h, w2_ref[...], preferred_element_type=F32) + b2_ref[...]
    h = jnp.maximum(h, 0.0)

    m2f = m2_ref[...]
    m2 = m2f > 0
    b2 = jnp.where(m2, batch_ref[...], G)
    iot = lax.broadcasted_iota(jnp.int32, (RB3, G + 1), 1)
    oh = b2 == iot
    def gmax(g, carry):
        row = jnp.max(jnp.where(b2 == g, h, NEG), axis=0, keepdims=True)
        gm_sc[pl.ds(g, 1), :] = jnp.maximum(gm_sc[pl.ds(g, 1), :], row)
        return carry

    lax.fori_loop(0, G, gmax, 0)

    t = t_ref[...]
    xlogy = jnp.where(t > 0, t * jnp.log(t), 0.0)
    klm = jnp.where(m2, xlogy - t * jnp.log(s2_ref[...] + 1e-14), 0.0)
    ohf = oh.astype(F32)
    kl_sc[...] += jnp.sum(ohf * klm, axis=0, keepdims=True)
    cnt_sc[...] += jnp.sum(ohf, axis=0, keepdims=True)
    rat_sc[...] += jnp.sum(m2f).reshape(1, 1)

    @pl.when(i == NB3 - 1)
    def _fin():
        gm = gm_sc[...][:G]
        pred_ref[...] = jnp.dot(gm, lw_ref[...],
                                preferred_element_type=F32) + lb_ref[...]
        cnt = jnp.maximum(cnt_sc[...][:, :G], 1.0)
        loss_ref[...] = kl_sc[...][:, :G] / cnt
        ratio_ref[...] = rat_sc[...] / jnp.float32(N)


_t3_call = None


def _t3(x2, agg, w1, b1, w2, b2, lw, lb, batchp, m2f, s2, tp):
    global _t3_call
    if _t3_call is None:
        row = lambda i: (i, 0)
        full = lambda i: (0, 0)
        _t3_call = pl.pallas_call(
            _t3_body,
            grid=(NB3,),
            in_specs=[
                pl.BlockSpec((RB3, HID), row),
                pl.BlockSpec((RB3, HID), row),
                pl.BlockSpec((HID, HID), full),
                pl.BlockSpec((1, HID), full),
                pl.BlockSpec((HID, HID), full),
                pl.BlockSpec((1, HID), full),
                pl.BlockSpec((HID, 1), full),
                pl.BlockSpec((1, 1), full),
                pl.BlockSpec((RB3, 1), row),
                pl.BlockSpec((RB3, 1), row),
                pl.BlockSpec((RB3, 1), row),
                pl.BlockSpec((RB3, 1), row),
            ],
            out_specs=[
                pl.BlockSpec((G, 1), full),
                pl.BlockSpec((1, G), full),
                pl.BlockSpec((1, 1), full),
            ],
            out_shape=[
                jax.ShapeDtypeStruct((G, 1), F32),
                jax.ShapeDtypeStruct((1, G), F32),
                jax.ShapeDtypeStruct((1, 1), F32),
            ],
            scratch_shapes=[
                pltpu.VMEM((G + 1, HID), F32),
                pltpu.VMEM((1, G + 1), F32),
                pltpu.VMEM((1, G + 1), F32),
                pltpu.VMEM((1, 1), F32),
            ],
        )
    return _t3_call(x2, agg, w1, b1.reshape(1, HID), w2,
                    b2.reshape(1, HID), lw, lb.reshape(1, 1), batchp, m2f,
                    s2, tp)


def kernel(x, edge_index, batch, node_attention,
           gin1_w1, gin1_b1, gin1_w2, gin1_b2, pool1_w, pool1_b,
           gin2_w1, gin2_b1, gin2_w2, gin2_b2, pool2_w, pool2_b,
           gin3_w1, gin3_b1, gin3_w2, gin3_b2, lin_w, lin_b):
    src = edge_index[0]
    dst = edge_index[1]
    dst2 = jnp.stack([jnp.where(dst < HALF, dst, JUNK),
                      jnp.where(dst >= HALF, dst - HALF, JUNK)])
    pad = NPAD - N
    xp = jnp.pad(x, ((0, pad), (0, 0)))
    batchp = jnp.pad(batch, (0, pad), constant_values=G).reshape(NPAD, 1)
    tp = jnp.pad(node_attention, (0, pad)).reshape(NPAD, 1)
    u1 = jnp.pad(jnp.ones((N,), F32), (0, pad)).reshape(NPAD, 1)
    z127 = jnp.zeros((NPAD, 127), F32)
    z63 = jnp.zeros((NPAD, 63), F32)
    z64 = jnp.zeros((NPAD, 64), F32)

    agg1 = _sc_scatter(xp, src, dst2)
    deg1 = _sc_scatter(jnp.concatenate([u1, z127], axis=1), src, dst2)
    h1, xw1 = _t1(xp, agg1, gin1_w1, gin1_b1, gin1_w2, gin1_b2, pool1_w)
    v1, dinv1 = _t2a(xw1, deg1[:, :1], u1)
    s1 = _sc_scatter(jnp.concatenate([v1, z127], axis=1), src, dst2)
    xk1, u2, _sk1 = _t2b(s1[:, :1], dinv1, u1, xw1, pool1_b, batchp, h1)

    agg2 = _sc_scatter(jnp.concatenate([xk1, u2, z63], axis=1), src, dst2)
    h2, xw2 = _t1(xk1, agg2[:, :HID],
                  gin2_w1, gin2_b1, gin2_w2, gin2_b2, pool2_w)
    v2, dinv2 = _t2a(xw2, agg2[:, HID:HID + 1], u2)
    s2 = _sc_scatter(jnp.concatenate([v2, z127], axis=1), src, dst2)
    xk2, u3, sk2 = _t2b(s2[:, :1], dinv2, u2, xw2, pool2_b, batchp, h2)

    agg3 = _sc_scatter(jnp.concatenate([xk2, z64], axis=1), src, dst2)
    pred, loss, ratio = _t3(xk2, agg3[:, :HID],
                            gin3_w1, gin3_b1, gin3_w2, gin3_b2,
                            lin_w, lin_b, batchp, u3, sk2, tp)
    return pred, loss.reshape(G), ratio.reshape(())


# speedup vs baseline: 11.5034x; 1.1139x over previous
import functools

import jax
import jax.numpy as jnp
from jax import lax
from jax.experimental import pallas as pl
from jax.experimental.pallas import tpu as pltpu
from jax.experimental.pallas import tpu_sc as plsc

N = 10000
NPAD = 10240
E = 320000
G = 64
DF = 128
HID = 64
NC = 2
NS = 16
NW = NC * NS
EW = E // NW
MIN_SCORE = 1e-3
TOL = 1e-7
NEG = float("-inf")
F32 = jnp.float32


HALF = NPAD // 2
JUNK = HALF
ACC_R = 5376
EW2 = E // NS
SC_D = 128
SC_C = 200
ZC = 48


@functools.lru_cache(maxsize=None)
def _make_sc_scatter():
    RW = ACC_R // NS
    mesh = plsc.VectorSubcoreMesh(core_axis_name="c", subcore_axis_name="s")

    @functools.partial(
        pl.kernel,
        out_type=jax.ShapeDtypeStruct((NC, ACC_R, SC_D), F32),
        mesh=mesh,
        scratch_types=[
            pltpu.VMEM((SC_C,), jnp.int32),
            pltpu.VMEM((SC_C,), jnp.int32),
            pltpu.VMEM((SC_C,), jnp.int32),
            pltpu.VMEM((SC_C,), jnp.int32),
            pltpu.VMEM((SC_C, SC_D), F32),
            pltpu.VMEM((SC_C, SC_D), F32),
            pltpu.VMEM((ZC, SC_D), F32),
            pltpu.VMEM_SHARED((ACC_R, SC_D), F32),
            pltpu.SemaphoreType.DMA,
            pltpu.SemaphoreType.DMA,
        ],
    )
    def scatter_kernel(table_hbm, src_hbm, dsta_hbm, dstb_hbm, zeros_hbm,
                       out_hbm, src_v0, dst_v0, src_v1, dst_v1, rows_v0,
                       rows_v1, zbuf_v, acc_s, sem0, sem1):
        c = lax.axis_index("c")
        s = lax.axis_index("s")

        pltpu.sync_copy(zeros_hbm, zbuf_v)

        @pl.loop(0, RW // ZC)
        def _zero(i):
            pltpu.sync_copy(zbuf_v, acc_s.at[pl.ds(s * RW + i * ZC, ZC)])

        plsc.subcore_barrier()

        base = s * EW2

        def run(dst_hbm):
            @pl.loop(0, EW2 // (2 * SC_C))
            def _scat(jj):
                off0 = base + 2 * jj * SC_C
                off1 = off0 + SC_C
                pltpu.sync_copy(src_hbm.at[pl.ds(off0, SC_C)], src_v0)
                pltpu.sync_copy(dst_hbm.at[pl.ds(off0, SC_C)], dst_v0)
                d0 = pltpu.async_copy(table_hbm.at[src_v0], rows_v0, sem0)
                pltpu.sync_copy(src_hbm.at[pl.ds(off1, SC_C)], src_v1)
                pltpu.sync_copy(dst_hbm.at[pl.ds(off1, SC_C)], dst_v1)
                d1 = pltpu.async_copy(table_hbm.at[src_v1], rows_v1, sem1)
                d0.wait()
                pltpu.sync_copy(rows_v0, acc_s.at[dst_v0], add=True)
                d1.wait()
                pltpu.sync_copy(rows_v1, acc_s.at[dst_v1], add=True)

        @pl.when(c == 0)
        def _ca():
            run(dsta_hbm)

        @pl.when(c == 1)
        def _cb():
            run(dstb_hbm)

        plsc.subcore_barrier()

        @pl.loop(0, RW // ZC)
        def _out(i):
            off = s * RW + i * ZC
            pltpu.sync_copy(acc_s.at[pl.ds(off, ZC)], zbuf_v)
            pltpu.sync_copy(zbuf_v, out_hbm.at[c, pl.ds(off, ZC)])

    return scatter_kernel


def _sc_scatter(table, src, dst2):
    halves = _make_sc_scatter()(table, src, dst2[0], dst2[1],
                                jnp.zeros((ZC, SC_D), F32))
    return jnp.concatenate([halves[0, :HALF], halves[1, :HALF]], axis=0)


RB = 1024


def _t1_body(x_ref, aa_ref, w1_ref, b1_ref, w2_ref, b2_ref, pw_ref,
             h_ref, xw_ref):
    h0 = x_ref[...] + aa_ref[...]
    h = jnp.maximum(jnp.dot(h0, w1_ref[...], preferred_element_type=F32)
                    + b1_ref[...], 0.0)
    h = jnp.dot(h, w2_ref[...], preferred_element_type=F32) + b2_ref[...]
    h = jnp.maximum(h, 0.0)
    h_ref[...] = h
    xw_ref[...] = jnp.dot(h, pw_ref[...], preferred_element_type=F32)


@functools.lru_cache(maxsize=None)
def _make_t1(din):
    grid = NPAD // RB
    row = lambda i: (i, 0)
    full = lambda i: (0, 0)
    return pl.pallas_call(
        _t1_body,
        grid=(grid,),
        in_specs=[
            pl.BlockSpec((RB, din), row),
            pl.BlockSpec((RB, din), row),
            pl.BlockSpec((din, HID), full),
            pl.BlockSpec((1, HID), full),
            pl.BlockSpec((HID, HID), full),
            pl.BlockSpec((1, HID), full),
            pl.BlockSpec((HID, 1), full),
        ],
        out_specs=[
            pl.BlockSpec((RB, HID), row),
            pl.BlockSpec((RB, 1), row),
        ],
        out_shape=[
            jax.ShapeDtypeStruct((NPAD, HID), F32),
            jax.ShapeDtypeStruct((NPAD, 1), F32),
        ],
    )


def _t1(x, agg, w1, b1, w2, b2, pw):
    return _make_t1(x.shape[1])(x, agg, w1, b1.reshape(1, HID),
                                w2, b2.reshape(1, HID), pw)


def _t2a_body(xw_ref, da_ref, u_ref, v_ref, dinv_ref):
    u = u_ref[...]
    deg = u * da_ref[...] + u
    dinv = jnp.where(deg > 0, 1.0 / jnp.sqrt(deg), 0.0)
    dinv_ref[...] = dinv
    v_ref[...] = dinv * u * xw_ref[...]


_t2a_call = None


def _t2a(xw, da, u):
    global _t2a_call
    if _t2a_call is None:
        _t2a_call = pl.pallas_call(
            _t2a_body,
            out_shape=[jax.ShapeDtypeStruct((NPAD, 1), F32),
                       jax.ShapeDtypeStruct((NPAD, 1), F32)],
        )
    return _t2a_call(xw, da, u)


CB2 = 1024
NCH2 = NPAD // CB2


def _t2b_body(sa_ref, dinv_ref, u_ref, xw_ref, pb_ref, batch_ref,
              h_ref, xk_ref, un_ref, sk_ref):
    iot = lax.broadcasted_iota(jnp.int32, (CB2, G + 1), 1)

    def chunk(i):
        sl = pl.ds(i * CB2, CB2)
        u = u_ref[sl, :]
        b = batch_ref[sl, :]
        dinv = dinv_ref[sl, :]
        oh = jnp.where(u > 0, b, G) == iot
        attn = (dinv * u * sa_ref[sl, :]
                + dinv * dinv * u * xw_ref[sl, :] + pb_ref[0, 0])
        return sl, u, b, oh, attn

    def p_max(i, m):
        _, _, _, oh, attn = chunk(i)
        return jnp.maximum(m, jnp.max(jnp.where(oh, attn, NEG), axis=0,
                                      keepdims=True))

    m = lax.fori_loop(0, NCH2, p_max, jnp.full((1, G + 1), NEG, F32))

    def score_chunk(i, d):
        sl, u, b, oh, attn = chunk(i)
        m_pn = jnp.sum(jnp.where(oh, m, 0.0), axis=1, keepdims=True)
        e = jnp.exp(attn - m_pn)
        if d is None:
            return sl, u, b, oh, e
        d_pn = jnp.sum(jnp.where(oh, d, 0.0), axis=1, keepdims=True)
        return sl, u, b, oh, e / d_pn

    def p_den(i, d):
        _, _, _, oh, e = score_chunk(i, None)
        return d + jnp.sum(jnp.where(oh, e, 0.0), axis=0, keepdims=True)

    d = lax.fori_loop(0, NCH2, p_den, jnp.zeros((1, G + 1), F32))

    def p_smax(i, smax):
        _, _, _, oh, score = score_chunk(i, d)
        return jnp.maximum(smax, jnp.max(jnp.where(oh, score, NEG), axis=0,
                                         keepdims=True))

    smax_seg = lax.fori_loop(0, NCH2, p_smax, jnp.full((1, G + 1), NEG, F32))

    def p_out(i, carry):
        sl, u, b, oh, score = score_chunk(i, d)
        oh_b = b == iot
        smax_pn = jnp.sum(jnp.where(oh_b, smax_seg, 0.0), axis=1,
                          keepdims=True) - TOL
        smin = jnp.minimum(smax_pn, MIN_SCORE)
        perm = (u > 0) & (score > smin)
        xk_ref[sl, :] = jnp.where(perm, h_ref[sl, :] * score, 0.0)
        un_ref[sl, :] = perm.astype(F32)
        sk_ref[sl, :] = jnp.where(perm, score, 0.0)
        return carry

    lax.fori_loop(0, NCH2, p_out, 0)


_t2b_call = None


def _t2b(sa, dinv, u, xw, pb, batchp, h):
    global _t2b_call
    if _t2b_call is None:
        _t2b_call = pl.pallas_call(
            _t2b_body,
            out_shape=[jax.ShapeDtypeStruct((NPAD, HID), F32),
                       jax.ShapeDtypeStruct((NPAD, 1), F32),
                       jax.ShapeDtypeStruct((NPAD, 1), F32)],
        )
    return _t2b_call(sa, dinv, u, xw, pb.reshape(1, 1), batchp, h)


RB3 = 256
NB3 = NPAD // RB3


def _t3_body(x_ref, aa_ref, w1_ref, b1_ref, w2_ref, b2_ref,
             lw_ref, lb_ref, batch_ref, m2_ref, s2_ref, t_ref,
             pred_ref, loss_ref, ratio_ref,
             gm_sc, kl_sc, cnt_sc, rat_sc):
    i = pl.program_id(0)

    @pl.when(i == 0)
    def _init():
        gm_sc[...] = jnp.full((G + 1, HID), NEG, F32)
        kl_sc[...] = jnp.zeros((1, G + 1), F32)
        cnt_sc[...] = jnp.zeros((1, G + 1), F32)
        rat_sc[...] = jnp.zeros((1, 1), F32)

    h0 = x_ref[...] + aa_ref[...]
    h = jnp.maximum(jnp.dot(h0, w1_ref[...], preferred_element_type=F32)
                    + b1_ref[...], 0.0)
    h = jnp.dot(h, w2_ref[...], preferred_element_type=F32) + b2_ref[...]
    h = jnp.maximum(h, 0.0)

    m2f = m2_ref[...]
    m2 = m2f > 0
    b2 = jnp.where(m2, batch_ref[...], G)
    iot = lax.broadcasted_iota(jnp.int32, (RB3, G + 1), 1)
    oh = b2 == iot
    def gmax(g, carry):
        row = jnp.max(jnp.where(b2 == g, h, NEG), axis=0, keepdims=True)
        gm_sc[pl.ds(g, 1), :] = jnp.maximum(gm_sc[pl.ds(g, 1), :], row)
        return carry

    lax.fori_loop(0, G, gmax, 0)

    t = t_ref[...]
    xlogy = jnp.where(t > 0, t * jnp.log(t), 0.0)
    klm = jnp.where(m2, xlogy - t * jnp.log(s2_ref[...] + 1e-14), 0.0)
    ohf = oh.astype(F32)
    kl_sc[...] += jnp.sum(ohf * klm, axis=0, keepdims=True)
    cnt_sc[...] += jnp.sum(ohf, axis=0, keepdims=True)
    rat_sc[...] += jnp.sum(m2f).reshape(1, 1)

    @pl.when(i == NB3 - 1)
    def _fin():
        gm = gm_sc[...][:G]
        pred_ref[...] = jnp.dot(gm, lw_ref[...],
                                preferred_element_type=F32) + lb_ref[...]
        cnt = jnp.maximum(cnt_sc[...][:, :G], 1.0)
        loss_ref[...] = kl_sc[...][:, :G] / cnt
        ratio_ref[...] = rat_sc[...] / jnp.float32(N)


_t3_call = None


def _t3(x2, agg, w1, b1, w2, b2, lw, lb, batchp, m2f, s2, tp):
    global _t3_call
    if _t3_call is None:
        row = lambda i: (i, 0)
        full = lambda i: (0, 0)
        _t3_call = pl.pallas_call(
            _t3_body,
            grid=(NB3,),
            in_specs=[
                pl.BlockSpec((RB3, HID), row),
                pl.BlockSpec((RB3, HID), row),
                pl.BlockSpec((HID, HID), full),
                pl.BlockSpec((1, HID), full),
                pl.BlockSpec((HID, HID), full),
                pl.BlockSpec((1, HID), full),
                pl.BlockSpec((HID, 1), full),
                pl.BlockSpec((1, 1), full),
                pl.BlockSpec((RB3, 1), row),
                pl.BlockSpec((RB3, 1), row),
                pl.BlockSpec((RB3, 1), row),
                pl.BlockSpec((RB3, 1), row),
            ],
            out_specs=[
                pl.BlockSpec((G, 1), full),
                pl.BlockSpec((1, G), full),
                pl.BlockSpec((1, 1), full),
            ],
            out_shape=[
                jax.ShapeDtypeStruct((G, 1), F32),
                jax.ShapeDtypeStruct((1, G), F32),
                jax.ShapeDtypeStruct((1, 1), F32),
            ],
            scratch_shapes=[
                pltpu.VMEM((G + 1, HID), F32),
                pltpu.VMEM((1, G + 1), F32),
                pltpu.VMEM((1, G + 1), F32),
                pltpu.VMEM((1, 1), F32),
            ],
        )
    return _t3_call(x2, agg, w1, b1.reshape(1, HID), w2,
                    b2.reshape(1, HID), lw, lb.reshape(1, 1), batchp, m2f,
                    s2, tp)


def kernel(x, edge_index, batch, node_attention,
           gin1_w1, gin1_b1, gin1_w2, gin1_b2, pool1_w, pool1_b,
           gin2_w1, gin2_b1, gin2_w2, gin2_b2, pool2_w, pool2_b,
           gin3_w1, gin3_b1, gin3_w2, gin3_b2, lin_w, lin_b):
    src = edge_index[0]
    dst = edge_index[1]
    dst2 = jnp.stack([jnp.where(dst < HALF, dst, JUNK),
                      jnp.where(dst >= HALF, dst - HALF, JUNK)])
    pad = NPAD - N
    xp = jnp.pad(x, ((0, pad), (0, 0)))
    batchp = jnp.pad(batch, (0, pad), constant_values=G).reshape(NPAD, 1)
    tp = jnp.pad(node_attention, (0, pad)).reshape(NPAD, 1)
    u1 = jnp.pad(jnp.ones((N,), F32), (0, pad)).reshape(NPAD, 1)
    z127 = jnp.zeros((NPAD, 127), F32)
    z63 = jnp.zeros((NPAD, 63), F32)
    z64 = jnp.zeros((NPAD, 64), F32)

    agg1 = _sc_scatter(xp, src, dst2)
    deg1 = _sc_scatter(jnp.concatenate([u1, z127], axis=1), src, dst2)
    h1, xw1 = _t1(xp, agg1, gin1_w1, gin1_b1, gin1_w2, gin1_b2, pool1_w)
    v1, dinv1 = _t2a(xw1, deg1[:, :1], u1)
    s1 = _sc_scatter(jnp.concatenate([v1, z127], axis=1), src, dst2)
    xk1, u2, _sk1 = _t2b(s1[:, :1], dinv1, u1, xw1, pool1_b, batchp, h1)

    agg2 = _sc_scatter(jnp.concatenate([xk1, u2, z63], axis=1), src, dst2)
    h2, xw2 = _t1(xk1, agg2[:, :HID],
                  gin2_w1, gin2_b1, gin2_w2, gin2_b2, pool2_w)
    v2, dinv2 = _t2a(xw2, agg2[:, HID:HID + 1], u2)
    s2 = _sc_scatter(jnp.concatenate([v2, z127], axis=1), src, dst2)
    xk2, u3, sk2 = _t2b(s2[:, :1], dinv2, u2, xw2, pool2_b, batchp, h2)

    agg3 = _sc_scatter(jnp.concatenate([xk2, z64], axis=1), src, dst2)
    pred, loss, ratio = _t3(xk2, agg3[:, :HID],
                            gin3_w1, gin3_b1, gin3_w2, gin3_b2,
                            lin_w, lin_b, batchp, u3, sk2, tp)
    return pred, loss.reshape(G), ratio.reshape(())
